# baseline (speedup 1.0000x reference)
import functools

import jax
import jax.numpy as jnp
from jax import lax
from jax.experimental import pallas as pl
from jax.experimental.pallas import tpu as pltpu
from jax.experimental.pallas import tpu_sc as plsc

N = 10000
E = 320000
D_EDGE = 16

NC = 2
NS = 16
NW = NC * NS
PER_W = E // NW
K = 80
NCHUNK = PER_W // K
RPT = N // NS

_mesh = functools.partial(
    plsc.VectorSubcoreMesh, core_axis_name="c", subcore_axis_name="s")
_sc_params = pltpu.CompilerParams(use_tc_tiling_on_sc=False)


def _chunks(SK):
  out = []
  o = 0
  while o < SK:
    k = min(K, SK - o)
    out.append((o, k))
    o += k
  return out


def _make_gather(C, SK):
  nsup = PER_W // SK
  plan = _chunks(SK)

  @functools.partial(
      pl.kernel, mesh=_mesh(), compiler_params=_sc_params,
      out_type=[jax.ShapeDtypeStruct((E, C), jnp.float32),
                jax.ShapeDtypeStruct((E, C), jnp.float32)],
      scratch_types=[
          pltpu.VMEM((SK,), jnp.int32),
          pltpu.VMEM((SK,), jnp.int32),
          pltpu.VMEM((SK, C), jnp.float32),
          pltpu.VMEM((SK, C), jnp.float32),
          pltpu.SemaphoreType.DMA,
          pltpu.SemaphoreType.DMA,
      ])
  def gat(xl_hbm, xr_hbm, src_hbm, dst_hbm, xlg_hbm, xrg_hbm,
          si_v, di_v, a_v, b_v, sem_a, sem_b):
    c = lax.axis_index("c")
    s = lax.axis_index("s")
    base = (c * NS + s) * PER_W

    def body(i, carry):
      off = base + i * SK
      pltpu.sync_copy(src_hbm.at[pl.ds(off, SK)], si_v)
      pltpu.sync_copy(dst_hbm.at[pl.ds(off, SK)], di_v)
      cps = []
      for (o, k) in plan:
        sl = pl.ds(o, k)
        cps.append(pltpu.async_copy(xl_hbm.at[si_v.at[sl]], a_v.at[sl], sem_a))
        cps.append(pltpu.async_copy(xr_hbm.at[di_v.at[sl]], b_v.at[sl], sem_b))
      for cp in cps:
        cp.wait()
      pltpu.sync_copy(a_v, xlg_hbm.at[pl.ds(off, SK)])
      pltpu.sync_copy(b_v, xrg_hbm.at[pl.ds(off, SK)])
      return carry

    lax.fori_loop(0, nsup, body, 0)

  return gat


def _make_scatter(W, SK):
  nsup = PER_W // SK
  plan = _chunks(SK)

  @functools.partial(
      pl.kernel, mesh=_mesh(), compiler_params=_sc_params,
      out_type=jax.ShapeDtypeStruct((NC, N, W), jnp.float32),
      scratch_types=[
          [pltpu.VMEM((k,), jnp.int32) for (_, k) in plan],
          pltpu.VMEM((SK, W), jnp.float32),
          pltpu.VMEM_SHARED((N, W), jnp.float32),
          pltpu.SemaphoreType.DMA,
      ])
  def sca(vals_hbm, dst_hbm, zeros_hbm, out_hbm, di_vs, v_v, acc_sh, sem_s):
    c = lax.axis_index("c")
    s = lax.axis_index("s")
    base = (c * NS + s) * PER_W
    row0 = s * RPT

    pltpu.sync_copy(zeros_hbm.at[pl.ds(row0, RPT)],
                    acc_sh.at[pl.ds(row0, RPT)])
    plsc.subcore_barrier()

    def body(i, carry):
      off = base + i * SK
      for t, (o, k) in enumerate(plan):
        pltpu.sync_copy(dst_hbm.at[pl.ds(off + o, k)], di_vs[t])
      pltpu.sync_copy(vals_hbm.at[pl.ds(off, SK)], v_v)
      cps = []
      for t, (o, k) in enumerate(plan):
        cps.append(pltpu.async_copy(
            v_v.at[pl.ds(o, k)], acc_sh.at[di_vs[t]], sem_s, add=True))
      for cp in cps:
        cp.wait()
      return carry

    lax.fori_loop(0, nsup, body, 0)
    plsc.subcore_barrier()

    pltpu.sync_copy(acc_sh.at[pl.ds(row0, RPT)],
                    out_hbm.at[c, pl.ds(row0, RPT)])

  return sca


_NB = 10
_NBLK = N // _NB
_EB = 80
_EBLK = E // _EB


def _mm_body(h_ref, wl_ref, bl_ref, wr_ref, br_ref, p0_ref, p1_ref,
             xl_ref, xr_ref, es_ref):
  h = h_ref[...]
  xl_ref[...] = jnp.dot(h, wl_ref[...],
                        preferred_element_type=jnp.float32) + bl_ref[...]
  xr_ref[...] = jnp.dot(h, wr_ref[...],
                        preferred_element_type=jnp.float32) + br_ref[...]
  p0 = p0_ref[...]
  p1 = p1_ref[...]
  esum = p0[:, :D_EDGE] + p1[:, :D_EDGE]
  deg = (p0[:, D_EDGE] + p1[:, D_EDGE])[:, None]
  es_ref[...] = esum / jnp.maximum(deg, 1.0)


def _node_transform(h, Wl, bl, Wr, br, p0, p1):
  din, C = Wl.shape
  return pl.pallas_call(
      _mm_body,
      grid=(_NB,),
      in_specs=[
          pl.BlockSpec((_NBLK, din), lambda i: (i, 0)),
          pl.BlockSpec((din, C), lambda i: (0, 0)),
          pl.BlockSpec((1, C), lambda i: (0, 0)),
          pl.BlockSpec((din, C), lambda i: (0, 0)),
          pl.BlockSpec((1, C), lambda i: (0, 0)),
          pl.BlockSpec((_NBLK, 48), lambda i: (i, 0)),
          pl.BlockSpec((_NBLK, 48), lambda i: (i, 0)),
      ],
      out_specs=[
          pl.BlockSpec((_NBLK, C), lambda i: (i, 0)),
          pl.BlockSpec((_NBLK, C), lambda i: (i, 0)),
          pl.BlockSpec((_NBLK, D_EDGE), lambda i: (i, 0)),
      ],
      out_shape=[jax.ShapeDtypeStruct((N, C), jnp.float32),
                 jax.ShapeDtypeStruct((N, C), jnp.float32),
                 jax.ShapeDtypeStruct((N, D_EDGE), jnp.float32)],
  )(h, Wl, bl.reshape(1, C), Wr, br.reshape(1, C), p0, p1)


def _edge_body(xlg_ref, xrg_ref, ea_ref, we_ref, att_ref, out_ref):
  xlg = xlg_ref[...]
  m = xlg + xrg_ref[...] + jnp.dot(ea_ref[...], we_ref[...],
                                   preferred_element_type=jnp.float32)
  m = jnp.where(m >= 0, m, 0.2 * m)
  alpha = jnp.sum(m * att_ref[...], axis=1, keepdims=True)
  ae = jnp.exp(alpha)
  out_ref[...] = jnp.concatenate(
      [ae * xlg, jnp.broadcast_to(ae, (ae.shape[0], 16))], axis=1)


def _edge_vals(xlg, xrg, ea, We, att):
  C = We.shape[1]
  W = C + 16
  return pl.pallas_call(
      _edge_body,
      grid=(_EB,),
      in_specs=[
          pl.BlockSpec((_EBLK, C), lambda i: (i, 0)),
          pl.BlockSpec((_EBLK, C), lambda i: (i, 0)),
          pl.BlockSpec((_EBLK, D_EDGE), lambda i: (i, 0)),
          pl.BlockSpec((D_EDGE, C), lambda i: (0, 0)),
          pl.BlockSpec((1, C), lambda i: (0, 0)),
      ],
      out_specs=pl.BlockSpec((_EBLK, W), lambda i: (i, 0)),
      out_shape=jax.ShapeDtypeStruct((E, W), jnp.float32),
  )(xlg, xrg, ea, We, att.reshape(1, C))


def _epi_body(p0_ref, p1_ref, xl_ref, xr_ref, es_ref, we_ref, att_ref,
              bias_ref, *rest, C, fused):
  xl = xl_ref[...]
  m = xl + xr_ref[...] + jnp.dot(es_ref[...], we_ref[...],
                                 preferred_element_type=jnp.float32)
  m = jnp.where(m >= 0, m, 0.2 * m)
  ae = jnp.exp(jnp.sum(m * att_ref[...], axis=1, keepdims=True))
  p0 = p0_ref[...]
  p1 = p1_ref[...]
  S = p0[:, :C] + p1[:, :C]
  asum = (p0[:, C] + p1[:, C])[:, None]
  h = (S + ae * xl) / (asum + ae + 1e-16) + bias_ref[...]
  h = jnp.maximum(h, 0.0)
  if not fused:
    (out_ref,) = rest
    out_ref[...] = h
  else:
    wl2_ref, bl2_ref, wr2_ref, br2_ref, xl2_ref, xr2_ref = rest
    xl2_ref[...] = jnp.dot(h, wl2_ref[...],
                           preferred_element_type=jnp.float32) + bl2_ref[...]
    xr2_ref[...] = jnp.dot(h, wr2_ref[...],
                           preferred_element_type=jnp.float32) + br2_ref[...]


def _epilogue(p0, p1, xl, xr, eself, We, att, bias, nxt=None):
  C = We.shape[1]
  W = C + 16
  in_specs = [
      pl.BlockSpec((_NBLK, W), lambda i: (i, 0)),
      pl.BlockSpec((_NBLK, W), lambda i: (i, 0)),
      pl.BlockSpec((_NBLK, C), lambda i: (i, 0)),
      pl.BlockSpec((_NBLK, C), lambda i: (i, 0)),
      pl.BlockSpec((_NBLK, D_EDGE), lambda i: (i, 0)),
      pl.BlockSpec((D_EDGE, C), lambda i: (0, 0)),
      pl.BlockSpec((1, C), lambda i: (0, 0)),
      pl.BlockSpec((1, C), lambda i: (0, 0)),
  ]
  args = [p0, p1, xl, xr, eself, We, att.reshape(1, C), bias.reshape(1, C)]
  if nxt is None:
    out_specs = pl.BlockSpec((_NBLK, C), lambda i: (i, 0))
    out_shape = jax.ShapeDtypeStruct((N, C), jnp.float32)
  else:
    Wl2, bl2, Wr2, br2 = nxt
    C2 = Wl2.shape[1]
    in_specs += [
        pl.BlockSpec((C, C2), lambda i: (0, 0)),
        pl.BlockSpec((1, C2), lambda i: (0, 0)),
        pl.BlockSpec((C, C2), lambda i: (0, 0)),
        pl.BlockSpec((1, C2), lambda i: (0, 0)),
    ]
    args += [Wl2, bl2.reshape(1, C2), Wr2, br2.reshape(1, C2)]
    out_specs = [pl.BlockSpec((_NBLK, C2), lambda i: (i, 0)),
                 pl.BlockSpec((_NBLK, C2), lambda i: (i, 0))]
    out_shape = [jax.ShapeDtypeStruct((N, C2), jnp.float32),
                 jax.ShapeDtypeStruct((N, C2), jnp.float32)]
  return pl.pallas_call(
      functools.partial(_epi_body, C=C, fused=nxt is not None),
      grid=(_NB,),
      in_specs=in_specs,
      out_specs=out_specs,
      out_shape=out_shape,
  )(*args)


def kernel(x, edge_index, edge_attr, params):
  src = edge_index[0]
  dst = edge_index[1]

  gather64 = _make_gather(64, 1000)
  gather32 = _make_gather(32, 1000)
  scatter80 = _make_scatter(80, 400)
  scatter48 = _make_scatter(48, 1000)
  gathers = {64: gather64, 32: gather32}
  scatters = {80: scatter80, 48: scatter48}

  pre_vals = jnp.concatenate(
      [edge_attr, jnp.ones((E, 16), jnp.float32),
       jnp.zeros((E, 16), jnp.float32)], axis=1)
  pre_acc = scatters[48](pre_vals, dst, jnp.zeros((N, 48), jnp.float32))

  Wl, bl, Wr, br, _, _, _ = params[0]
  xl, xr, eself = _node_transform(x, Wl, bl, Wr, br, pre_acc[0], pre_acc[1])

  for i, (Wl, bl, Wr, br, We, att, bias) in enumerate(params):
    C = Wl.shape[1]
    W = C + 16
    xlg, xrg = gathers[C](xl, xr, src, dst)
    vals = _edge_vals(xlg, xrg, edge_attr, We, att)
    acc = scatters[W](vals, dst, jnp.zeros((N, W), jnp.float32))
    if i + 1 < len(params):
      nxt = params[i + 1][:4]
      xl, xr = _epilogue(acc[0], acc[1], xl, xr, eself, We, att, bias,
                         nxt=nxt)
    else:
      h = _epilogue(acc[0], acc[1], xl, xr, eself, We, att, bias)
  return h


# speedup vs baseline: 1.0384x; 1.0384x over previous
import functools

import jax
import jax.numpy as jnp
from jax import lax
from jax.experimental import pallas as pl
from jax.experimental.pallas import tpu as pltpu
from jax.experimental.pallas import tpu_sc as plsc

N = 10000
E = 320000
D_EDGE = 16

NC = 2
NS = 16
NW = NC * NS
PER_W = E // NW
K = 80
NCHUNK = PER_W // K
RPT = N // NS

_mesh = functools.partial(
    plsc.VectorSubcoreMesh, core_axis_name="c", subcore_axis_name="s")
_sc_params = pltpu.CompilerParams(use_tc_tiling_on_sc=False)


def _chunks(SK):
  out = []
  o = 0
  while o < SK:
    k = min(K, SK - o)
    out.append((o, k))
    o += k
  return out


def _make_gather(C, SK):
  nsup = PER_W // SK
  plan = _chunks(SK)

  @functools.partial(
      pl.kernel, mesh=_mesh(), compiler_params=_sc_params,
      out_type=[jax.ShapeDtypeStruct((E, C), jnp.float32),
                jax.ShapeDtypeStruct((E, C), jnp.float32)],
      scratch_types=[
          pltpu.VMEM((SK,), jnp.int32),
          pltpu.VMEM((SK,), jnp.int32),
          pltpu.VMEM((SK, C), jnp.float32),
          pltpu.VMEM((SK, C), jnp.float32),
          pltpu.SemaphoreType.DMA,
          pltpu.SemaphoreType.DMA,
      ])
  def gat(xl_hbm, xr_hbm, src_hbm, dst_hbm, xlg_hbm, xrg_hbm,
          si_v, di_v, a_v, b_v, sem_a, sem_b):
    c = lax.axis_index("c")
    s = lax.axis_index("s")
    base = (c * NS + s) * PER_W

    def body(i, carry):
      off = base + i * SK
      pltpu.sync_copy(src_hbm.at[pl.ds(off, SK)], si_v)
      pltpu.sync_copy(dst_hbm.at[pl.ds(off, SK)], di_v)
      cps = []
      for (o, k) in plan:
        sl = pl.ds(o, k)
        cps.append(pltpu.async_copy(xl_hbm.at[si_v.at[sl]], a_v.at[sl], sem_a))
        cps.append(pltpu.async_copy(xr_hbm.at[di_v.at[sl]], b_v.at[sl], sem_b))
      for cp in cps:
        cp.wait()
      pltpu.sync_copy(a_v, xlg_hbm.at[pl.ds(off, SK)])
      pltpu.sync_copy(b_v, xrg_hbm.at[pl.ds(off, SK)])
      return carry

    lax.fori_loop(0, nsup, body, 0)

  return gat


def _make_scatter(W, SK):
  nsup = PER_W // SK
  plan = _chunks(SK)

  @functools.partial(
      pl.kernel, mesh=_mesh(), compiler_params=_sc_params,
      out_type=jax.ShapeDtypeStruct((NC, N, W), jnp.float32),
      scratch_types=[
          [pltpu.VMEM((k,), jnp.int32) for (_, k) in plan],
          pltpu.VMEM((SK, W), jnp.float32),
          pltpu.VMEM_SHARED((N, W), jnp.float32),
          pltpu.SemaphoreType.DMA,
      ])
  def sca(vals_hbm, dst_hbm, zeros_hbm, out_hbm, di_vs, v_v, acc_sh, sem_s):
    c = lax.axis_index("c")
    s = lax.axis_index("s")
    base = (c * NS + s) * PER_W
    row0 = s * RPT

    pltpu.sync_copy(zeros_hbm.at[pl.ds(row0, RPT)],
                    acc_sh.at[pl.ds(row0, RPT)])
    plsc.subcore_barrier()

    def body(i, carry):
      off = base + i * SK
      for t, (o, k) in enumerate(plan):
        pltpu.sync_copy(dst_hbm.at[pl.ds(off + o, k)], di_vs[t])
      pltpu.sync_copy(vals_hbm.at[pl.ds(off, SK)], v_v)
      cps = []
      for t, (o, k) in enumerate(plan):
        cps.append(pltpu.async_copy(
            v_v.at[pl.ds(o, k)], acc_sh.at[di_vs[t]], sem_s, add=True))
      for cp in cps:
        cp.wait()
      return carry

    lax.fori_loop(0, nsup, body, 0)
    plsc.subcore_barrier()

    pltpu.sync_copy(acc_sh.at[pl.ds(row0, RPT)],
                    out_hbm.at[c, pl.ds(row0, RPT)])

  return sca


_NB = 10
_NBLK = N // _NB
_EB = 80
_EBLK = E // _EB


def _mm_body(h_ref, wl_ref, bl_ref, wr_ref, br_ref, xl_ref, xr_ref):
  h = h_ref[...]
  xl_ref[...] = jnp.dot(h, wl_ref[...],
                        preferred_element_type=jnp.float32) + bl_ref[...]
  xr_ref[...] = jnp.dot(h, wr_ref[...],
                        preferred_element_type=jnp.float32) + br_ref[...]


def _node_transform(h, Wl, bl, Wr, br):
  din, C = Wl.shape
  return pl.pallas_call(
      _mm_body,
      grid=(_NB,),
      in_specs=[
          pl.BlockSpec((_NBLK, din), lambda i: (i, 0)),
          pl.BlockSpec((din, C), lambda i: (0, 0)),
          pl.BlockSpec((1, C), lambda i: (0, 0)),
          pl.BlockSpec((din, C), lambda i: (0, 0)),
          pl.BlockSpec((1, C), lambda i: (0, 0)),
      ],
      out_specs=[
          pl.BlockSpec((_NBLK, C), lambda i: (i, 0)),
          pl.BlockSpec((_NBLK, C), lambda i: (i, 0)),
      ],
      out_shape=[jax.ShapeDtypeStruct((N, C), jnp.float32),
                 jax.ShapeDtypeStruct((N, C), jnp.float32)],
  )(h, Wl, bl.reshape(1, C), Wr, br.reshape(1, C))


def _edge_body(xlg_ref, xrg_ref, ea_ref, we_ref, att_ref, out_ref):
  xlg = xlg_ref[...]
  m = xlg + xrg_ref[...] + jnp.dot(ea_ref[...], we_ref[...],
                                   preferred_element_type=jnp.float32)
  m = jnp.where(m >= 0, m, 0.2 * m)
  alpha = jnp.sum(m * att_ref[...], axis=1, keepdims=True)
  ae = jnp.exp(alpha)
  out_ref[...] = jnp.concatenate(
      [ae * xlg, jnp.broadcast_to(ae, (ae.shape[0], 16))], axis=1)


def _edge_vals(xlg, xrg, ea, We, att):
  C = We.shape[1]
  W = C + 16
  return pl.pallas_call(
      _edge_body,
      grid=(_EB,),
      in_specs=[
          pl.BlockSpec((_EBLK, C), lambda i: (i, 0)),
          pl.BlockSpec((_EBLK, C), lambda i: (i, 0)),
          pl.BlockSpec((_EBLK, D_EDGE), lambda i: (i, 0)),
          pl.BlockSpec((D_EDGE, C), lambda i: (0, 0)),
          pl.BlockSpec((1, C), lambda i: (0, 0)),
      ],
      out_specs=pl.BlockSpec((_EBLK, W), lambda i: (i, 0)),
      out_shape=jax.ShapeDtypeStruct((E, W), jnp.float32),
  )(xlg, xrg, ea, We, att.reshape(1, C))


def _epi_body(p0_ref, p1_ref, xl_ref, xr_ref, es_ref, we_ref, att_ref,
              bias_ref, out_ref, *, C):
  xl = xl_ref[...]
  m = xl + xr_ref[...] + jnp.dot(es_ref[...], we_ref[...],
                                 preferred_element_type=jnp.float32)
  m = jnp.where(m >= 0, m, 0.2 * m)
  ae = jnp.exp(jnp.sum(m * att_ref[...], axis=1, keepdims=True))
  p0 = p0_ref[...]
  p1 = p1_ref[...]
  S = p0[:, :C] + p1[:, :C]
  asum = (p0[:, C] + p1[:, C])[:, None]
  out = (S + ae * xl) / (asum + ae + 1e-16) + bias_ref[...]
  out_ref[...] = jnp.maximum(out, 0.0)


def _epilogue(p0, p1, xl, xr, eself, We, att, bias):
  C = We.shape[1]
  W = C + 16
  return pl.pallas_call(
      functools.partial(_epi_body, C=C),
      grid=(_NB,),
      in_specs=[
          pl.BlockSpec((_NBLK, W), lambda i: (i, 0)),
          pl.BlockSpec((_NBLK, W), lambda i: (i, 0)),
          pl.BlockSpec((_NBLK, C), lambda i: (i, 0)),
          pl.BlockSpec((_NBLK, C), lambda i: (i, 0)),
          pl.BlockSpec((_NBLK, D_EDGE), lambda i: (i, 0)),
          pl.BlockSpec((D_EDGE, C), lambda i: (0, 0)),
          pl.BlockSpec((1, C), lambda i: (0, 0)),
          pl.BlockSpec((1, C), lambda i: (0, 0)),
      ],
      out_specs=pl.BlockSpec((_NBLK, C), lambda i: (i, 0)),
      out_shape=jax.ShapeDtypeStruct((N, C), jnp.float32),
  )(p0, p1, xl, xr, eself, We, att.reshape(1, C), bias.reshape(1, C))


def _eself_body(p0_ref, p1_ref, out_ref):
  p0 = p0_ref[...]
  p1 = p1_ref[...]
  esum = p0[:, :D_EDGE] + p1[:, :D_EDGE]
  deg = (p0[:, D_EDGE] + p1[:, D_EDGE])[:, None]
  out_ref[...] = esum / jnp.maximum(deg, 1.0)


def _eself(p0, p1):
  W = 48
  return pl.pallas_call(
      _eself_body,
      grid=(_NB,),
      in_specs=[
          pl.BlockSpec((_NBLK, W), lambda i: (i, 0)),
          pl.BlockSpec((_NBLK, W), lambda i: (i, 0)),
      ],
      out_specs=pl.BlockSpec((_NBLK, D_EDGE), lambda i: (i, 0)),
      out_shape=jax.ShapeDtypeStruct((N, D_EDGE), jnp.float32),
  )(p0, p1)


def kernel(x, edge_index, edge_attr, params):
  src = edge_index[0]
  dst = edge_index[1]

  gather64 = _make_gather(64, 1000)
  gather32 = _make_gather(32, 1000)
  scatter80 = _make_scatter(80, 400)
  scatter48 = _make_scatter(48, 1000)
  gathers = {64: gather64, 32: gather32}
  scatters = {80: scatter80, 48: scatter48}

  pre_vals = jnp.concatenate(
      [edge_attr, jnp.ones((E, 16), jnp.float32),
       jnp.zeros((E, 16), jnp.float32)], axis=1)
  pre_acc = scatters[48](pre_vals, dst, jnp.zeros((N, 48), jnp.float32))
  eself = _eself(pre_acc[0], pre_acc[1])

  h = x
  for (Wl, bl, Wr, br, We, att, bias) in params:
    C = Wl.shape[1]
    W = C + 16
    xl, xr = _node_transform(h, Wl, bl, Wr, br)
    xlg, xrg = gathers[C](xl, xr, src, dst)
    vals = _edge_vals(xlg, xrg, edge_attr, We, att)
    acc = scatters[W](vals, dst, jnp.zeros((N, W), jnp.float32))
    h = _epilogue(acc[0], acc[1], xl, xr, eself, We, att, bias)
  return h
